# Initial kernel scaffold; baseline (speedup 1.0000x reference)
import jax
import jax.numpy as jnp
from jax.experimental import pallas as pl


def kernel(x, adj, W1, W2, W3, W4):
    raise NotImplementedError("write your pallas kernel here")


# speedup vs baseline: 1.0263x; 1.0263x over previous
import jax
import jax.numpy as jnp
from jax.experimental import pallas as pl

NN = 10000


def _proj_body(h_ref, w_ref, y_ref):
    y_ref[...] = jnp.dot(
        h_ref[...].astype(jnp.bfloat16),
        w_ref[...].astype(jnp.bfloat16),
        preferred_element_type=jnp.float32,
    ).astype(jnp.bfloat16)


def _proj(h, w):
    return pl.pallas_call(
        _proj_body,
        out_shape=jax.ShapeDtypeStruct((NN, w.shape[1]), jnp.bfloat16),
    )(h, w)


def _layer1_body(adj_ref, y_ref, abf_ref, out_ref):
    ab = adj_ref[...].astype(jnp.bfloat16)
    abf_ref[...] = ab
    z = jnp.dot(ab, y_ref[...], preferred_element_type=jnp.float32)
    out_ref[...] = jnp.maximum(z, 0.0)


def _layer1(adj, y):
    bm = 200
    e = y.shape[1]
    return pl.pallas_call(
        _layer1_body,
        grid=(NN // bm,),
        in_specs=[
            pl.BlockSpec((bm, NN), lambda i: (i, 0)),
            pl.BlockSpec((NN, e), lambda i: (0, 0)),
        ],
        out_specs=[
            pl.BlockSpec((bm, NN), lambda i: (i, 0)),
            pl.BlockSpec((bm, e), lambda i: (i, 0)),
        ],
        out_shape=[
            jax.ShapeDtypeStruct((NN, NN), jnp.bfloat16),
            jax.ShapeDtypeStruct((NN, e), jnp.float32),
        ],
    )(adj, y)


def _layer_body(abf_ref, y_ref, out_ref):
    z = jnp.dot(abf_ref[...], y_ref[...], preferred_element_type=jnp.float32)
    out_ref[...] = jnp.maximum(z, 0.0)


def _layer(abf, y):
    bm = 400
    e = y.shape[1]
    return pl.pallas_call(
        _layer_body,
        grid=(NN // bm,),
        in_specs=[
            pl.BlockSpec((bm, NN), lambda i: (i, 0)),
            pl.BlockSpec((NN, e), lambda i: (0, 0)),
        ],
        out_specs=pl.BlockSpec((bm, e), lambda i: (i, 0)),
        out_shape=jax.ShapeDtypeStruct((NN, e), jnp.float32),
    )(abf, y)


def _norm_body(h4_ref, h_ref, hbf_ref):
    v = h4_ref[...]
    n = jnp.maximum(jnp.sqrt(jnp.sum(v * v, axis=1, keepdims=True)), 1e-12)
    h = v / n
    h_ref[...] = h
    hbf_ref[...] = h.astype(jnp.bfloat16)


def _norm(h4):
    e = h4.shape[1]
    return pl.pallas_call(
        _norm_body,
        out_shape=[
            jax.ShapeDtypeStruct((NN, e), jnp.float32),
            jax.ShapeDtypeStruct((NN, e), jnp.bfloat16),
        ],
    )(h4)


def _ahat_body(hblk_ref, hfull_ref, out_ref):
    t = jax.lax.dot_general(
        hblk_ref[...],
        hfull_ref[...],
        (((1,), (1,)), ((), ())),
        preferred_element_type=jnp.float32,
    )
    out_ref[...] = jax.nn.sigmoid(t)


def _ahat(hbf):
    bm = 200
    e = hbf.shape[1]
    return pl.pallas_call(
        _ahat_body,
        grid=(NN // bm,),
        in_specs=[
            pl.BlockSpec((bm, e), lambda i: (i, 0)),
            pl.BlockSpec((NN, e), lambda i: (0, 0)),
        ],
        out_specs=pl.BlockSpec((bm, NN), lambda i: (i, 0)),
        out_shape=jax.ShapeDtypeStruct((NN, NN), jnp.float32),
    )(hbf, hbf)


def kernel(x, adj, W1, W2, W3, W4):
    y1 = _proj(x, W1)
    adj_bf, enc_h1 = _layer1(adj, y1)
    enc_h2 = _layer(adj_bf, _proj(enc_h1, W2))
    enc_h3 = _layer(adj_bf, _proj(enc_h2, W3))
    enc_h4 = _layer(adj_bf, _proj(enc_h3, W4))
    h, hbf = _norm(enc_h4)
    a_hat = _ahat(hbf)
    return (enc_h1, enc_h2, enc_h3, enc_h4, h, a_hat)


# speedup vs baseline: 1.1446x; 1.1153x over previous
import jax
import jax.numpy as jnp
from jax.experimental import pallas as pl
from jax.experimental.pallas import tpu as pltpu

NN = 10000
BF = jnp.bfloat16


def _layer1_body(x_ref, w1_ref, w2_ref, adj_ref, abf_ref, h1_ref, y2_ref, y1_s):
    @pl.when(pl.program_id(0) == 0)
    def _():
        y1_s[...] = jnp.dot(
            x_ref[...].astype(BF), w1_ref[...].astype(BF),
            preferred_element_type=jnp.float32,
        ).astype(BF)

    ab = adj_ref[...].astype(BF)
    abf_ref[...] = ab
    z = jnp.maximum(
        jnp.dot(ab, y1_s[...], preferred_element_type=jnp.float32), 0.0)
    h1_ref[...] = z
    y2_ref[...] = jnp.dot(
        z.astype(BF), w2_ref[...].astype(BF),
        preferred_element_type=jnp.float32,
    ).astype(BF)


def _layer1(x, adj, w1, w2):
    bm = 200
    e_in, e_out = w1.shape[1], w2.shape[1]
    return pl.pallas_call(
        _layer1_body,
        grid=(NN // bm,),
        in_specs=[
            pl.BlockSpec((NN, w1.shape[0]), lambda i: (0, 0)),
            pl.BlockSpec(w1.shape, lambda i: (0, 0)),
            pl.BlockSpec(w2.shape, lambda i: (0, 0)),
            pl.BlockSpec((bm, NN), lambda i: (i, 0)),
        ],
        out_specs=[
            pl.BlockSpec((bm, NN), lambda i: (i, 0)),
            pl.BlockSpec((bm, e_in), lambda i: (i, 0)),
            pl.BlockSpec((bm, e_out), lambda i: (i, 0)),
        ],
        out_shape=[
            jax.ShapeDtypeStruct((NN, NN), BF),
            jax.ShapeDtypeStruct((NN, e_in), jnp.float32),
            jax.ShapeDtypeStruct((NN, e_out), BF),
        ],
        scratch_shapes=[pltpu.VMEM((NN, e_in), BF)],
    )(x, w1, w2, adj)


def _mid_layer_body(abf_ref, y_ref, w_ref, h_ref, ynext_ref):
    z = jnp.maximum(
        jnp.dot(abf_ref[...], y_ref[...], preferred_element_type=jnp.float32),
        0.0)
    h_ref[...] = z
    ynext_ref[...] = jnp.dot(
        z.astype(BF), w_ref[...].astype(BF),
        preferred_element_type=jnp.float32,
    ).astype(BF)


def _mid_layer(abf, y, w):
    bm = 1000
    e_in, e_out = y.shape[1], w.shape[1]
    return pl.pallas_call(
        _mid_layer_body,
        grid=(NN // bm,),
        in_specs=[
            pl.BlockSpec((bm, NN), lambda i: (i, 0)),
            pl.BlockSpec((NN, e_in), lambda i: (0, 0)),
            pl.BlockSpec(w.shape, lambda i: (0, 0)),
        ],
        out_specs=[
            pl.BlockSpec((bm, e_in), lambda i: (i, 0)),
            pl.BlockSpec((bm, e_out), lambda i: (i, 0)),
        ],
        out_shape=[
            jax.ShapeDtypeStruct((NN, e_in), jnp.float32),
            jax.ShapeDtypeStruct((NN, e_out), BF),
        ],
    )(abf, y, w)


def _last_layer_body(abf_ref, y_ref, h4_ref, h_ref, hbf_ref):
    z = jnp.maximum(
        jnp.dot(abf_ref[...], y_ref[...], preferred_element_type=jnp.float32),
        0.0)
    h4_ref[...] = z
    n = jnp.maximum(jnp.sqrt(jnp.sum(z * z, axis=1, keepdims=True)), 1e-12)
    h = z / n
    h_ref[...] = h
    hbf_ref[...] = h.astype(BF)


def _last_layer(abf, y):
    bm = 1000
    e = y.shape[1]
    return pl.pallas_call(
        _last_layer_body,
        grid=(NN // bm,),
        in_specs=[
            pl.BlockSpec((bm, NN), lambda i: (i, 0)),
            pl.BlockSpec((NN, e), lambda i: (0, 0)),
        ],
        out_specs=[
            pl.BlockSpec((bm, e), lambda i: (i, 0)),
            pl.BlockSpec((bm, e), lambda i: (i, 0)),
            pl.BlockSpec((bm, e), lambda i: (i, 0)),
        ],
        out_shape=[
            jax.ShapeDtypeStruct((NN, e), jnp.float32),
            jax.ShapeDtypeStruct((NN, e), jnp.float32),
            jax.ShapeDtypeStruct((NN, e), BF),
        ],
    )(abf, y)


def _ahat_body(hblk_ref, hfull_ref, out_ref):
    t = jax.lax.dot_general(
        hblk_ref[...], hfull_ref[...],
        (((1,), (1,)), ((), ())),
        preferred_element_type=jnp.float32,
    )
    out_ref[...] = 0.5 * jnp.tanh(0.5 * t) + 0.5


def _ahat(hbf):
    bm = 400
    e = hbf.shape[1]
    return pl.pallas_call(
        _ahat_body,
        grid=(NN // bm,),
        in_specs=[
            pl.BlockSpec((bm, e), lambda i: (i, 0)),
            pl.BlockSpec((NN, e), lambda i: (0, 0)),
        ],
        out_specs=pl.BlockSpec((bm, NN), lambda i: (i, 0)),
        out_shape=jax.ShapeDtypeStruct((NN, NN), jnp.float32),
    )(hbf, hbf)


def kernel(x, adj, W1, W2, W3, W4):
    adj_bf, enc_h1, y2 = _layer1(x, adj, W1, W2)
    enc_h2, y3 = _mid_layer(adj_bf, y2, W3)
    enc_h3, y4 = _mid_layer(adj_bf, y3, W4)
    enc_h4, h, hbf = _last_layer(adj_bf, y4)
    a_hat = _ahat(hbf)
    return (enc_h1, enc_h2, enc_h3, enc_h4, h, a_hat)
